# baseline (speedup 1.0000x reference)
import functools

import jax
import jax.numpy as jnp
from jax import lax
from jax.experimental import pallas as pl
from jax.experimental.pallas import tpu as pltpu
from jax.experimental.pallas import tpu_sc as plsc


def _make_sc_gather(n, c):
    info = plsc.get_sparse_core_info()
    num_cores = info.num_cores
    num_subcores = info.num_subcores
    lanes = info.num_lanes
    nw = num_cores * num_subcores
    bpw = n // nw

    mesh = plsc.VectorSubcoreMesh(core_axis_name="c", subcore_axis_name="s")

    @functools.partial(
        pl.kernel,
        mesh=mesh,
        out_type=jax.ShapeDtypeStruct((n,), jnp.float32),
        scratch_types=[
            pltpu.VMEM((bpw,), jnp.int32),
            pltpu.VMEM((bpw,), jnp.int32),
            pltpu.VMEM((bpw,), jnp.float32),
            pltpu.SemaphoreType.DMA,
        ],
    )
    def sc_gather(x_hbm, tgt_hbm, out_hbm, tgt_v, idx_v, val_v, sem):
        wid = lax.axis_index("s") * num_cores + lax.axis_index("c")
        base = wid * bpw
        pltpu.sync_copy(tgt_hbm.at[pl.ds(base, bpw)], tgt_v)
        for k in range(bpw // lanes):
            sl = pl.ds(k * lanes, lanes)
            rows = (base + k * lanes) + lax.broadcasted_iota(
                jnp.int32, (lanes,), 0
            )
            idx_v[sl] = tgt_v[sl] + rows * c
        pltpu.async_copy(x_hbm.at[idx_v], val_v, sem).wait()
        pltpu.sync_copy(val_v, out_hbm.at[pl.ds(base, bpw)])

    return sc_gather


def _make_tc_loss(n, c, rb, cb):
    nrb = n // rb
    ncb = -(-c // cb)

    def body(x_ref, ratio_ref, tgt_ref, out_ref, m_ref, s_ref, acc_ref):
        i = pl.program_id(0)
        j = pl.program_id(1)

        @pl.when(j == 0)
        def _init():
            m_ref[...] = jnp.full((rb, 1), -jnp.inf, jnp.float32)
            s_ref[...] = jnp.zeros((rb, 1), jnp.float32)

        m_prev = m_ref[...]
        s_prev = s_ref[...]

        def update(xb):
            m_new = jnp.maximum(m_prev, jnp.max(xb, axis=1, keepdims=True))
            s_new = s_prev * jnp.exp(m_prev - m_new) + jnp.sum(
                jnp.exp(xb - m_new), axis=1, keepdims=True
            )
            return m_new, s_new

        @pl.when(j < ncb - 1)
        def _mid():
            m_new, s_new = update(x_ref[...])
            m_ref[...] = m_new
            s_ref[...] = s_new

        @pl.when(j == ncb - 1)
        def _last():
            cols = lax.broadcasted_iota(jnp.int32, (rb, cb), 1) + j * cb
            xm = jnp.where(cols < c, x_ref[...], -jnp.inf)
            m_new, s_new = update(xm)
            lse = m_new + jnp.log(s_new)
            contrib = jnp.sum(
                ratio_ref[...] * (tgt_ref[...] - lse), axis=0, keepdims=True
            )

            @pl.when(i == 0)
            def _():
                acc_ref[...] = contrib

            @pl.when(i > 0)
            def _():
                acc_ref[...] = acc_ref[...] + contrib

            @pl.when(i == nrb - 1)
            def _():
                out_ref[...] = -acc_ref[...] / n

    return pl.pallas_call(
        body,
        grid=(nrb, ncb),
        in_specs=[
            pl.BlockSpec((rb, cb), lambda i, j: (i, j)),
            pl.BlockSpec((rb, 1), lambda i, j: (i, 0)),
            pl.BlockSpec((rb, 1), lambda i, j: (i, 0)),
        ],
        out_specs=pl.BlockSpec((1, 1), lambda i, j: (0, 0)),
        out_shape=jax.ShapeDtypeStruct((1, 1), jnp.float32),
        scratch_shapes=[
            pltpu.VMEM((rb, 1), jnp.float32),
            pltpu.VMEM((rb, 1), jnp.float32),
            pltpu.VMEM((1, 1), jnp.float32),
        ],
    )


@jax.jit
def kernel(ratio, inputs, targets):
    n, c = inputs.shape
    tgt = targets.astype(jnp.int32)
    x_flat = inputs.reshape(-1)
    tgt_vals = _make_sc_gather(n, c)(x_flat, tgt)
    loss = _make_tc_loss(n, c, rb=256, cb=2048)(
        inputs, ratio, tgt_vals.reshape(n, 1)
    )
    return loss.reshape(())


# speedup vs baseline: 1.7781x; 1.7781x over previous
import functools

import numpy as np

import jax
import jax.numpy as jnp
from jax import lax
from jax.experimental import pallas as pl
from jax.experimental.pallas import tpu as pltpu
from jax.experimental.pallas import tpu_sc as plsc


def _make_tc_pass(n, c, rb):
    nrb = n // rb

    def body(x_ref, tgt_ref, lse_ref, tval_ref):
        x = x_ref[...]
        m = jnp.max(x, axis=1, keepdims=True)
        s = jnp.sum(jnp.exp(x - m), axis=1, keepdims=True)
        lse_ref[...] = m + jnp.log(s)
        eq = lax.broadcasted_iota(jnp.int32, (rb, c), 1) == tgt_ref[...]
        tval_ref[...] = jnp.sum(jnp.where(eq, x, 0.0), axis=1, keepdims=True)

    return pl.pallas_call(
        body,
        grid=(nrb,),
        in_specs=[
            pl.BlockSpec((rb, c), lambda i: (i, 0)),
            pl.BlockSpec((rb, 1), lambda i: (i, 0)),
        ],
        out_specs=[
            pl.BlockSpec((rb, 1), lambda i: (i, 0)),
            pl.BlockSpec((rb, 1), lambda i: (i, 0)),
        ],
        out_shape=[
            jax.ShapeDtypeStruct((n, 1), jnp.float32),
            jax.ShapeDtypeStruct((n, 1), jnp.float32),
        ],
    )


def _make_sc_combine(n):
    info = plsc.get_sparse_core_info()
    lanes = info.num_lanes
    mesh = plsc.VectorSubcoreMesh(core_axis_name="c", subcore_axis_name="s")

    @functools.partial(
        pl.kernel,
        mesh=mesh,
        out_type=jax.ShapeDtypeStruct((lanes,), jnp.float32),
        scratch_types=[
            pltpu.VMEM((n,), jnp.float32),
            pltpu.VMEM((n,), jnp.float32),
            pltpu.VMEM((n,), jnp.float32),
            pltpu.VMEM((lanes,), jnp.float32),
        ],
    )
    def sc_combine(ratio_hbm, tval_hbm, lse_hbm, out_hbm, r_v, t_v, l_v, o_v):
        wid = lax.axis_index("s") * info.num_cores + lax.axis_index("c")

        @pl.when(wid == 0)
        def _():
            pltpu.sync_copy(ratio_hbm, r_v)
            pltpu.sync_copy(tval_hbm, t_v)
            pltpu.sync_copy(lse_hbm, l_v)
            acc = jnp.zeros((lanes,), jnp.float32)
            for k in range(n // lanes):
                sl = pl.ds(k * lanes, lanes)
                acc = acc + r_v[sl] * (t_v[sl] - l_v[sl])
            lane_ids = lax.broadcasted_iota(jnp.int32, (lanes,), 0)
            dnums = lax.GatherDimensionNumbers(
                offset_dims=(),
                collapsed_slice_dims=(0,),
                start_index_map=(0,),
            )
            step = 1
            while step < lanes:
                perm = (lane_ids ^ step).reshape(lanes, 1)
                acc = acc + lax.gather(
                    acc,
                    perm,
                    dnums,
                    (1,),
                    mode=lax.GatherScatterMode.PROMISE_IN_BOUNDS,
                )
                step *= 2
            o_v[...] = acc * (-1.0 / n)
            pltpu.sync_copy(o_v, out_hbm)

    return sc_combine


@jax.jit
def kernel(ratio, inputs, targets):
    n, c = inputs.shape
    tgt = targets.astype(jnp.int32).reshape(n, 1)
    lse, tval = _make_tc_pass(n, c, rb=8)(inputs, tgt)
    out = _make_sc_combine(n)(
        ratio.reshape(n), tval.reshape(n), lse.reshape(n)
    )
    return out[0]


# speedup vs baseline: 5.7925x; 3.2577x over previous
import functools

import numpy as np

import jax
import jax.numpy as jnp
from jax import lax
from jax.experimental import pallas as pl
from jax.experimental.pallas import tpu as pltpu
from jax.experimental.pallas import tpu_sc as plsc


def _make_tc_pass(n, c, kb):
    ncb = c // kb

    def body(x_ref, tgt_ref, lse_ref, tval_ref, m_ref, s_ref, tv_ref):
        j = pl.program_id(0)

        @pl.when(j == 0)
        def _init():
            m_ref[...] = jnp.full((1, n), -jnp.inf, jnp.float32)
            s_ref[...] = jnp.zeros((1, n), jnp.float32)
            tv_ref[...] = jnp.zeros((1, n), jnp.float32)

        x = x_ref[...]
        m_prev = m_ref[...]
        m_new = jnp.maximum(m_prev, jnp.max(x, axis=0, keepdims=True))
        s_ref[...] = s_ref[...] * jnp.exp(m_prev - m_new) + jnp.sum(
            jnp.exp(x - m_new), axis=0, keepdims=True
        )
        m_ref[...] = m_new

        cls = lax.broadcasted_iota(jnp.int32, (kb, n), 0) + j * kb
        eq = cls == tgt_ref[...]
        tv_ref[...] = tv_ref[...] + jnp.sum(
            jnp.where(eq, x, 0.0), axis=0, keepdims=True
        )

        @pl.when(j == ncb - 1)
        def _fin():
            lse_ref[...] = m_ref[...] + jnp.log(s_ref[...])
            tval_ref[...] = tv_ref[...]

    return pl.pallas_call(
        body,
        grid=(ncb,),
        in_specs=[
            pl.BlockSpec((kb, n), lambda j: (j, 0)),
            pl.BlockSpec((1, n), lambda j: (0, 0)),
        ],
        out_specs=[
            pl.BlockSpec((1, n), lambda j: (0, 0)),
            pl.BlockSpec((1, n), lambda j: (0, 0)),
        ],
        out_shape=[
            jax.ShapeDtypeStruct((1, n), jnp.float32),
            jax.ShapeDtypeStruct((1, n), jnp.float32),
        ],
        scratch_shapes=[
            pltpu.VMEM((1, n), jnp.float32),
            pltpu.VMEM((1, n), jnp.float32),
            pltpu.VMEM((1, n), jnp.float32),
        ],
    )


def _make_sc_combine(n):
    info = plsc.get_sparse_core_info()
    lanes = info.num_lanes
    mesh = plsc.VectorSubcoreMesh(core_axis_name="c", subcore_axis_name="s")

    @functools.partial(
        pl.kernel,
        mesh=mesh,
        out_type=jax.ShapeDtypeStruct((lanes,), jnp.float32),
        scratch_types=[
            pltpu.VMEM((n,), jnp.float32),
            pltpu.VMEM((n,), jnp.float32),
            pltpu.VMEM((n,), jnp.float32),
            pltpu.VMEM((lanes,), jnp.float32),
        ],
    )
    def sc_combine(ratio_hbm, tval_hbm, lse_hbm, out_hbm, r_v, t_v, l_v, o_v):
        wid = lax.axis_index("s") * info.num_cores + lax.axis_index("c")

        @pl.when(wid == 0)
        def _():
            pltpu.sync_copy(ratio_hbm, r_v)
            pltpu.sync_copy(tval_hbm, t_v)
            pltpu.sync_copy(lse_hbm, l_v)
            acc = jnp.zeros((lanes,), jnp.float32)
            for k in range(n // lanes):
                sl = pl.ds(k * lanes, lanes)
                acc = acc + r_v[sl] * (t_v[sl] - l_v[sl])
            lane_ids = lax.broadcasted_iota(jnp.int32, (lanes,), 0)
            dnums = lax.GatherDimensionNumbers(
                offset_dims=(),
                collapsed_slice_dims=(0,),
                start_index_map=(0,),
            )
            step = 1
            while step < lanes:
                perm = (lane_ids ^ step).reshape(lanes, 1)
                acc = acc + lax.gather(
                    acc,
                    perm,
                    dnums,
                    (1,),
                    mode=lax.GatherScatterMode.PROMISE_IN_BOUNDS,
                )
                step *= 2
            o_v[...] = acc * (-1.0 / n)
            pltpu.sync_copy(o_v, out_hbm)

    return sc_combine


@jax.jit
def kernel(ratio, inputs, targets):
    n, c = inputs.shape
    xt = jnp.swapaxes(inputs, 0, 1)
    tgt = targets.astype(jnp.int32).reshape(1, n)
    lse, tval = _make_tc_pass(n, c, kb=1000)(xt, tgt)
    out = _make_sc_combine(n)(
        ratio.reshape(n), tval.reshape(n), lse.reshape(n)
    )
    return out[0]


# speedup vs baseline: 6.4912x; 1.1206x over previous
import functools

import numpy as np

import jax
import jax.numpy as jnp
from jax import lax
from jax.experimental import pallas as pl
from jax.experimental.pallas import tpu as pltpu
from jax.experimental.pallas import tpu_sc as plsc


def _make_tc_pass(n, c, kb):
    ncb = c // kb

    def body(x_ref, tgt_ref, lse_ref, tval_ref, m_ref, s_ref, tv_ref):
        j = pl.program_id(0)

        @pl.when(j == 0)
        def _init():
            m_ref[...] = jnp.full((1, n), -jnp.inf, jnp.float32)
            s_ref[...] = jnp.zeros((1, n), jnp.float32)
            tv_ref[...] = jnp.zeros((1, n), jnp.float32)

        x = x_ref[...]
        m_prev = m_ref[...]
        m_new = jnp.maximum(m_prev, jnp.max(x, axis=0, keepdims=True))
        s_ref[...] = s_ref[...] * jnp.exp(m_prev - m_new) + jnp.sum(
            jnp.exp(x - m_new), axis=0, keepdims=True
        )
        m_ref[...] = m_new

        cls = lax.broadcasted_iota(jnp.int32, (kb, n), 0) + j * kb
        eq = cls == tgt_ref[...]
        tv_ref[...] = tv_ref[...] + jnp.sum(
            jnp.where(eq, x, 0.0), axis=0, keepdims=True
        )

        @pl.when(j == ncb - 1)
        def _fin():
            lse_ref[...] = m_ref[...] + jnp.log(s_ref[...])
            tval_ref[...] = tv_ref[...]

    return pl.pallas_call(
        body,
        grid=(ncb,),
        in_specs=[
            pl.BlockSpec((kb, n), lambda j: (j, 0)),
            pl.BlockSpec((1, n), lambda j: (0, 0)),
        ],
        out_specs=[
            pl.BlockSpec((1, n), lambda j: (0, 0)),
            pl.BlockSpec((1, n), lambda j: (0, 0)),
        ],
        out_shape=[
            jax.ShapeDtypeStruct((1, n), jnp.float32),
            jax.ShapeDtypeStruct((1, n), jnp.float32),
        ],
        scratch_shapes=[
            pltpu.VMEM((1, n), jnp.float32),
            pltpu.VMEM((1, n), jnp.float32),
            pltpu.VMEM((1, n), jnp.float32),
        ],
    )


def _make_sc_combine(n):
    info = plsc.get_sparse_core_info()
    lanes = info.num_lanes
    mesh = plsc.VectorSubcoreMesh(core_axis_name="c", subcore_axis_name="s")

    @functools.partial(
        pl.kernel,
        mesh=mesh,
        out_type=jax.ShapeDtypeStruct((lanes,), jnp.float32),
        scratch_types=[
            pltpu.VMEM((n,), jnp.float32),
            pltpu.VMEM((n,), jnp.float32),
            pltpu.VMEM((n,), jnp.float32),
            pltpu.VMEM((lanes,), jnp.float32),
        ],
    )
    def sc_combine(ratio_hbm, tval_hbm, lse_hbm, out_hbm, r_v, t_v, l_v, o_v):
        wid = lax.axis_index("s") * info.num_cores + lax.axis_index("c")

        @pl.when(wid == 0)
        def _():
            pltpu.sync_copy(ratio_hbm, r_v)
            pltpu.sync_copy(tval_hbm, t_v)
            pltpu.sync_copy(lse_hbm, l_v)
            acc = jnp.zeros((lanes,), jnp.float32)
            for k in range(n // lanes):
                sl = pl.ds(k * lanes, lanes)
                acc = acc + r_v[sl] * (t_v[sl] - l_v[sl])
            lane_ids = lax.broadcasted_iota(jnp.int32, (lanes,), 0)
            dnums = lax.GatherDimensionNumbers(
                offset_dims=(),
                collapsed_slice_dims=(0,),
                start_index_map=(0,),
            )
            step = 1
            while step < lanes:
                perm = (lane_ids ^ step).reshape(lanes, 1)
                acc = acc + lax.gather(
                    acc,
                    perm,
                    dnums,
                    (1,),
                    mode=lax.GatherScatterMode.PROMISE_IN_BOUNDS,
                )
                step *= 2
            o_v[...] = acc * (-1.0 / n)
            pltpu.sync_copy(o_v, out_hbm)

    return sc_combine


@jax.jit
def kernel(ratio, inputs, targets):
    n, c = inputs.shape
    xt = jnp.swapaxes(inputs, 0, 1)
    tgt = targets.astype(jnp.int32).reshape(1, n)
    lse, tval = _make_tc_pass(n, c, kb=2000)(xt, tgt)
    out = _make_sc_combine(n)(
        ratio.reshape(n), tval.reshape(n), lse.reshape(n)
    )
    return out[0]


# speedup vs baseline: 6.7364x; 1.0378x over previous
import functools

import numpy as np

import jax
import jax.numpy as jnp
from jax import lax
from jax.experimental import pallas as pl
from jax.experimental.pallas import tpu as pltpu
from jax.experimental.pallas import tpu_sc as plsc


def _make_tc_pass(n, c, kb):
    ncb = c // kb

    def body(x_ref, tgt_ref, lse_ref, tval_ref, m_ref, s_ref, tv_ref):
        j = pl.program_id(0)

        @pl.when(j == 0)
        def _init():
            m_ref[...] = jnp.full((1, n), -jnp.inf, jnp.float32)
            s_ref[...] = jnp.zeros((1, n), jnp.float32)
            tv_ref[...] = jnp.zeros((1, n), jnp.float32)

        x = x_ref[...]
        m_prev = m_ref[...]
        m_new = jnp.maximum(m_prev, jnp.max(x, axis=0, keepdims=True))
        s_ref[...] = s_ref[...] * jnp.exp(m_prev - m_new) + jnp.sum(
            jnp.exp(x - m_new), axis=0, keepdims=True
        )
        m_ref[...] = m_new

        cls = lax.broadcasted_iota(jnp.int32, (kb, n), 0) + j * kb
        eq = cls == tgt_ref[...]
        tv_ref[...] = tv_ref[...] + jnp.sum(
            jnp.where(eq, x, 0.0), axis=0, keepdims=True
        )

        @pl.when(j == ncb - 1)
        def _fin():
            lse_ref[...] = m_ref[...] + jnp.log(s_ref[...])
            tval_ref[...] = tv_ref[...]

    return pl.pallas_call(
        body,
        grid=(ncb,),
        in_specs=[
            pl.BlockSpec((kb, n), lambda j: (j, 0)),
            pl.BlockSpec((1, n), lambda j: (0, 0)),
        ],
        out_specs=[
            pl.BlockSpec((1, n), lambda j: (0, 0)),
            pl.BlockSpec((1, n), lambda j: (0, 0)),
        ],
        out_shape=[
            jax.ShapeDtypeStruct((1, n), jnp.float32),
            jax.ShapeDtypeStruct((1, n), jnp.float32),
        ],
        scratch_shapes=[
            pltpu.VMEM((1, n), jnp.float32),
            pltpu.VMEM((1, n), jnp.float32),
            pltpu.VMEM((1, n), jnp.float32),
        ],
    )


def _make_sc_combine(n):
    info = plsc.get_sparse_core_info()
    lanes = info.num_lanes
    mesh = plsc.VectorSubcoreMesh(core_axis_name="c", subcore_axis_name="s")

    @functools.partial(
        pl.kernel,
        mesh=mesh,
        out_type=jax.ShapeDtypeStruct((lanes,), jnp.float32),
        scratch_types=[
            pltpu.VMEM((n,), jnp.float32),
            pltpu.VMEM((n,), jnp.float32),
            pltpu.VMEM((n,), jnp.float32),
            pltpu.VMEM((lanes,), jnp.float32),
        ],
    )
    def sc_combine(ratio_hbm, tval_hbm, lse_hbm, out_hbm, r_v, t_v, l_v, o_v):
        wid = lax.axis_index("s") * info.num_cores + lax.axis_index("c")

        @pl.when(wid == 0)
        def _():
            pltpu.sync_copy(ratio_hbm, r_v)
            pltpu.sync_copy(tval_hbm, t_v)
            pltpu.sync_copy(lse_hbm, l_v)
            acc = jnp.zeros((lanes,), jnp.float32)
            for k in range(n // lanes):
                sl = pl.ds(k * lanes, lanes)
                acc = acc + r_v[sl] * (t_v[sl] - l_v[sl])
            lane_ids = lax.broadcasted_iota(jnp.int32, (lanes,), 0)
            dnums = lax.GatherDimensionNumbers(
                offset_dims=(),
                collapsed_slice_dims=(0,),
                start_index_map=(0,),
            )
            step = 1
            while step < lanes:
                perm = (lane_ids ^ step).reshape(lanes, 1)
                acc = acc + lax.gather(
                    acc,
                    perm,
                    dnums,
                    (1,),
                    mode=lax.GatherScatterMode.PROMISE_IN_BOUNDS,
                )
                step *= 2
            o_v[...] = acc * (-1.0 / n)
            pltpu.sync_copy(o_v, out_hbm)

    return sc_combine


@jax.jit
def kernel(ratio, inputs, targets):
    n, c = inputs.shape
    xt = jnp.swapaxes(inputs, 0, 1)
    tgt = targets.astype(jnp.int32).reshape(1, n)
    lse, tval = _make_tc_pass(n, c, kb=4000)(xt, tgt)
    out = _make_sc_combine(n)(
        ratio.reshape(n), tval.reshape(n), lse.reshape(n)
    )
    return out[0]


# speedup vs baseline: 6.7636x; 1.0040x over previous
import functools

import numpy as np

import jax
import jax.numpy as jnp
from jax import lax
from jax.experimental import pallas as pl
from jax.experimental.pallas import tpu as pltpu
from jax.experimental.pallas import tpu_sc as plsc


def _make_tc_pass(n, c, kb):
    ncb = c // kb

    def body(x_ref, tgt_ref, lse_ref, tval_ref, m_ref, s_ref, tv_ref):
        j = pl.program_id(0)

        @pl.when(j == 0)
        def _init():
            m_ref[...] = jnp.full((1, n), -jnp.inf, jnp.float32)
            s_ref[...] = jnp.zeros((1, n), jnp.float32)
            tv_ref[...] = jnp.zeros((1, n), jnp.float32)

        x = x_ref[...]
        m_prev = m_ref[...]
        m_new = jnp.maximum(m_prev, jnp.max(x, axis=0, keepdims=True))
        s_ref[...] = s_ref[...] * jnp.exp(m_prev - m_new) + jnp.sum(
            jnp.exp(x - m_new), axis=0, keepdims=True
        )
        m_ref[...] = m_new

        cls = lax.broadcasted_iota(jnp.int32, (kb, n), 0) + j * kb
        eq = cls == tgt_ref[...]
        tv_ref[...] = tv_ref[...] + jnp.sum(
            jnp.where(eq, x, 0.0), axis=0, keepdims=True
        )

        @pl.when(j == ncb - 1)
        def _fin():
            lse_ref[...] = m_ref[...] + jnp.log(s_ref[...])
            tval_ref[...] = tv_ref[...]

    return pl.pallas_call(
        body,
        grid=(ncb,),
        in_specs=[
            pl.BlockSpec((kb, n), lambda j: (j, 0)),
            pl.BlockSpec((1, n), lambda j: (0, 0)),
        ],
        out_specs=[
            pl.BlockSpec((1, n), lambda j: (0, 0)),
            pl.BlockSpec((1, n), lambda j: (0, 0)),
        ],
        out_shape=[
            jax.ShapeDtypeStruct((1, n), jnp.float32),
            jax.ShapeDtypeStruct((1, n), jnp.float32),
        ],
        scratch_shapes=[
            pltpu.VMEM((1, n), jnp.float32),
            pltpu.VMEM((1, n), jnp.float32),
            pltpu.VMEM((1, n), jnp.float32),
        ],
    )


def _make_sc_combine(n):
    info = plsc.get_sparse_core_info()
    lanes = info.num_lanes
    mesh = plsc.VectorSubcoreMesh(core_axis_name="c", subcore_axis_name="s")

    @functools.partial(
        pl.kernel,
        mesh=mesh,
        out_type=jax.ShapeDtypeStruct((lanes,), jnp.float32),
        scratch_types=[
            pltpu.VMEM((n,), jnp.float32),
            pltpu.VMEM((n,), jnp.float32),
            pltpu.VMEM((n,), jnp.float32),
            pltpu.VMEM((lanes,), jnp.float32),
        ],
    )
    def sc_combine(ratio_hbm, tval_hbm, lse_hbm, out_hbm, r_v, t_v, l_v, o_v):
        wid = lax.axis_index("s") * info.num_cores + lax.axis_index("c")

        @pl.when(wid == 0)
        def _():
            pltpu.sync_copy(ratio_hbm, r_v)
            pltpu.sync_copy(tval_hbm, t_v)
            pltpu.sync_copy(lse_hbm, l_v)
            acc = jnp.zeros((lanes,), jnp.float32)
            for k in range(n // lanes):
                sl = pl.ds(k * lanes, lanes)
                acc = acc + r_v[sl] * (t_v[sl] - l_v[sl])
            lane_ids = lax.broadcasted_iota(jnp.int32, (lanes,), 0)
            dnums = lax.GatherDimensionNumbers(
                offset_dims=(),
                collapsed_slice_dims=(0,),
                start_index_map=(0,),
            )
            step = 1
            while step < lanes:
                perm = (lane_ids ^ step).reshape(lanes, 1)
                acc = acc + lax.gather(
                    acc,
                    perm,
                    dnums,
                    (1,),
                    mode=lax.GatherScatterMode.PROMISE_IN_BOUNDS,
                )
                step *= 2
            o_v[...] = acc * (-1.0 / n)
            pltpu.sync_copy(o_v, out_hbm)

    return sc_combine


@jax.jit
def kernel(ratio, inputs, targets):
    n, c = inputs.shape
    xt = jnp.swapaxes(inputs, 0, 1)
    tgt = targets.astype(jnp.int32).reshape(1, n)
    lse, tval = _make_tc_pass(n, c, kb=5000)(xt, tgt)
    out = _make_sc_combine(n)(
        ratio.reshape(n), tval.reshape(n), lse.reshape(n)
    )
    return out[0]


# speedup vs baseline: 6.9230x; 1.0236x over previous
import functools

import numpy as np

import jax
import jax.numpy as jnp
from jax import lax
from jax.experimental import pallas as pl
from jax.experimental.pallas import tpu as pltpu
from jax.experimental.pallas import tpu_sc as plsc


def _make_tc_pass(n, c, kb):
    ncb = c // kb

    def body(x_ref, tgt_ref, lse_ref, tval_ref, m_ref, s_ref, tv_ref):
        j = pl.program_id(0)

        @pl.when(j == 0)
        def _init():
            m_ref[...] = jnp.full((1, n), -jnp.inf, jnp.float32)
            s_ref[...] = jnp.zeros((1, n), jnp.float32)
            tv_ref[...] = jnp.zeros((1, n), jnp.float32)

        x = x_ref[...]
        m_prev = m_ref[...]
        m_new = jnp.maximum(m_prev, jnp.max(x, axis=0, keepdims=True))
        s_ref[...] = s_ref[...] * jnp.exp(m_prev - m_new) + jnp.sum(
            jnp.exp(x - m_new), axis=0, keepdims=True
        )
        m_ref[...] = m_new

        eq = lax.broadcasted_iota(jnp.int32, (kb, n), 0) == tgt_ref[...] - j * kb
        tv_ref[...] = tv_ref[...] + jnp.sum(
            jnp.where(eq, x, 0.0), axis=0, keepdims=True
        )

        @pl.when(j == ncb - 1)
        def _fin():
            lse_ref[...] = m_ref[...] + jnp.log(s_ref[...])
            tval_ref[...] = tv_ref[...]

    return pl.pallas_call(
        body,
        grid=(ncb,),
        in_specs=[
            pl.BlockSpec((kb, n), lambda j: (j, 0)),
            pl.BlockSpec((1, n), lambda j: (0, 0)),
        ],
        out_specs=[
            pl.BlockSpec((1, n), lambda j: (0, 0)),
            pl.BlockSpec((1, n), lambda j: (0, 0)),
        ],
        out_shape=[
            jax.ShapeDtypeStruct((1, n), jnp.float32),
            jax.ShapeDtypeStruct((1, n), jnp.float32),
        ],
        scratch_shapes=[
            pltpu.VMEM((1, n), jnp.float32),
            pltpu.VMEM((1, n), jnp.float32),
            pltpu.VMEM((1, n), jnp.float32),
        ],
    )


def _make_sc_combine(n):
    info = plsc.get_sparse_core_info()
    lanes = info.num_lanes
    mesh = plsc.VectorSubcoreMesh(core_axis_name="c", subcore_axis_name="s")

    @functools.partial(
        pl.kernel,
        mesh=mesh,
        out_type=jax.ShapeDtypeStruct((lanes,), jnp.float32),
        scratch_types=[
            pltpu.VMEM((n,), jnp.float32),
            pltpu.VMEM((n,), jnp.float32),
            pltpu.VMEM((n,), jnp.float32),
            pltpu.VMEM((lanes,), jnp.float32),
        ],
    )
    def sc_combine(ratio_hbm, tval_hbm, lse_hbm, out_hbm, r_v, t_v, l_v, o_v):
        wid = lax.axis_index("s") * info.num_cores + lax.axis_index("c")

        @pl.when(wid == 0)
        def _():
            pltpu.sync_copy(ratio_hbm, r_v)
            pltpu.sync_copy(tval_hbm, t_v)
            pltpu.sync_copy(lse_hbm, l_v)
            acc = jnp.zeros((lanes,), jnp.float32)
            for k in range(n // lanes):
                sl = pl.ds(k * lanes, lanes)
                acc = acc + r_v[sl] * (t_v[sl] - l_v[sl])
            lane_ids = lax.broadcasted_iota(jnp.int32, (lanes,), 0)
            dnums = lax.GatherDimensionNumbers(
                offset_dims=(),
                collapsed_slice_dims=(0,),
                start_index_map=(0,),
            )
            step = 1
            while step < lanes:
                perm = (lane_ids ^ step).reshape(lanes, 1)
                acc = acc + lax.gather(
                    acc,
                    perm,
                    dnums,
                    (1,),
                    mode=lax.GatherScatterMode.PROMISE_IN_BOUNDS,
                )
                step *= 2
            o_v[...] = acc * (-1.0 / n)
            pltpu.sync_copy(o_v, out_hbm)

    return sc_combine


@jax.jit
def kernel(ratio, inputs, targets):
    n, c = inputs.shape
    xt = jnp.swapaxes(inputs, 0, 1)
    tgt = targets.astype(jnp.int32).reshape(1, n)
    lse, tval = _make_tc_pass(n, c, kb=5000)(xt, tgt)
    out = _make_sc_combine(n)(
        ratio.reshape(n), tval.reshape(n), lse.reshape(n)
    )
    return out[0]


# speedup vs baseline: 7.0372x; 1.0165x over previous
import functools

import numpy as np

import jax
import jax.numpy as jnp
from jax import lax
from jax.experimental import pallas as pl
from jax.experimental.pallas import tpu as pltpu
from jax.experimental.pallas import tpu_sc as plsc


def _make_tc_pass(n, c, kb, nstream):
    ncb = c // (kb * nstream)

    def body(*refs):
        x_refs = refs[:nstream]
        tgt_ref = refs[nstream]
        lse_ref, tval_ref = refs[nstream + 1], refs[nstream + 2]
        m_ref, s_ref, tv_ref = refs[nstream + 3:]
        j = pl.program_id(0)

        @pl.when(j == 0)
        def _init():
            m_ref[...] = jnp.full((1, n), -jnp.inf, jnp.float32)
            s_ref[...] = jnp.zeros((1, n), jnp.float32)
            tv_ref[...] = jnp.zeros((1, n), jnp.float32)

        xs = [r[...] for r in x_refs]
        m_new = m_ref[...]
        for x in xs:
            m_new = jnp.maximum(m_new, jnp.max(x, axis=0, keepdims=True))
        s_new = s_ref[...] * jnp.exp(m_ref[...] - m_new)
        tv_new = tv_ref[...]
        iota = lax.broadcasted_iota(jnp.int32, (kb, n), 0)
        for t, x in enumerate(xs):
            s_new = s_new + jnp.sum(jnp.exp(x - m_new), axis=0, keepdims=True)
            eq = iota == tgt_ref[...] - (j * nstream + t) * kb
            tv_new = tv_new + jnp.sum(
                jnp.where(eq, x, 0.0), axis=0, keepdims=True
            )
        m_ref[...] = m_new
        s_ref[...] = s_new
        tv_ref[...] = tv_new

        @pl.when(j == ncb - 1)
        def _fin():
            lse_ref[...] = m_ref[...] + jnp.log(s_ref[...])
            tval_ref[...] = tv_ref[...]

    def x_spec(t):
        return pl.BlockSpec((kb, n), lambda j, t=t: (j * nstream + t, 0))

    return pl.pallas_call(
        body,
        grid=(ncb,),
        in_specs=[x_spec(t) for t in range(nstream)]
        + [pl.BlockSpec((1, n), lambda j: (0, 0))],
        out_specs=[
            pl.BlockSpec((1, n), lambda j: (0, 0)),
            pl.BlockSpec((1, n), lambda j: (0, 0)),
        ],
        out_shape=[
            jax.ShapeDtypeStruct((1, n), jnp.float32),
            jax.ShapeDtypeStruct((1, n), jnp.float32),
        ],
        scratch_shapes=[
            pltpu.VMEM((1, n), jnp.float32),
            pltpu.VMEM((1, n), jnp.float32),
            pltpu.VMEM((1, n), jnp.float32),
        ],
    )


def _make_sc_combine(n):
    info = plsc.get_sparse_core_info()
    lanes = info.num_lanes
    mesh = plsc.VectorSubcoreMesh(core_axis_name="c", subcore_axis_name="s")

    @functools.partial(
        pl.kernel,
        mesh=mesh,
        out_type=jax.ShapeDtypeStruct((lanes,), jnp.float32),
        scratch_types=[
            pltpu.VMEM((n,), jnp.float32),
            pltpu.VMEM((n,), jnp.float32),
            pltpu.VMEM((n,), jnp.float32),
            pltpu.VMEM((lanes,), jnp.float32),
        ],
    )
    def sc_combine(ratio_hbm, tval_hbm, lse_hbm, out_hbm, r_v, t_v, l_v, o_v):
        wid = lax.axis_index("s") * info.num_cores + lax.axis_index("c")

        @pl.when(wid == 0)
        def _():
            pltpu.sync_copy(ratio_hbm, r_v)
            pltpu.sync_copy(tval_hbm, t_v)
            pltpu.sync_copy(lse_hbm, l_v)
            acc = jnp.zeros((lanes,), jnp.float32)
            for k in range(n // lanes):
                sl = pl.ds(k * lanes, lanes)
                acc = acc + r_v[sl] * (t_v[sl] - l_v[sl])
            lane_ids = lax.broadcasted_iota(jnp.int32, (lanes,), 0)
            dnums = lax.GatherDimensionNumbers(
                offset_dims=(),
                collapsed_slice_dims=(0,),
                start_index_map=(0,),
            )
            step = 1
            while step < lanes:
                perm = (lane_ids ^ step).reshape(lanes, 1)
                acc = acc + lax.gather(
                    acc,
                    perm,
                    dnums,
                    (1,),
                    mode=lax.GatherScatterMode.PROMISE_IN_BOUNDS,
                )
                step *= 2
            o_v[...] = acc * (-1.0 / n)
            pltpu.sync_copy(o_v, out_hbm)

    return sc_combine


@jax.jit
def kernel(ratio, inputs, targets):
    n, c = inputs.shape
    xt = jnp.swapaxes(inputs, 0, 1)
    tgt = targets.astype(jnp.int32).reshape(1, n)
    nstream = 2
    lse, tval = _make_tc_pass(n, c, kb=2000, nstream=nstream)(
        *([xt] * nstream), tgt
    )
    out = _make_sc_combine(n)(
        ratio.reshape(n), tval.reshape(n), lse.reshape(n)
    )
    return out[0]


# speedup vs baseline: 7.1189x; 1.0116x over previous
import functools

import numpy as np

import jax
import jax.numpy as jnp
from jax import lax
from jax.experimental import pallas as pl
from jax.experimental.pallas import tpu as pltpu
from jax.experimental.pallas import tpu_sc as plsc


def _make_tc_pass(n, c, kb, nstream):
    ncb = c // (kb * nstream)

    def body(*refs):
        x_refs = refs[:nstream]
        tgt_ref = refs[nstream]
        lse_ref, tval_ref = refs[nstream + 1], refs[nstream + 2]
        m_ref, s_ref, tv_ref = refs[nstream + 3:]
        j = pl.program_id(0)

        @pl.when(j == 0)
        def _init():
            m_ref[...] = jnp.full((1, n), -jnp.inf, jnp.float32)
            s_ref[...] = jnp.zeros((1, n), jnp.float32)
            tv_ref[...] = jnp.zeros((1, n), jnp.float32)

        xs = [r[...] for r in x_refs]
        m_new = m_ref[...]
        for x in xs:
            m_new = jnp.maximum(m_new, jnp.max(x, axis=0, keepdims=True))
        s_new = s_ref[...] * jnp.exp(m_ref[...] - m_new)
        tv_new = tv_ref[...]
        iota = lax.broadcasted_iota(jnp.int32, (kb, n), 0)
        for t, x in enumerate(xs):
            s_new = s_new + jnp.sum(jnp.exp(x - m_new), axis=0, keepdims=True)
            eq = iota == tgt_ref[...] - (j * nstream + t) * kb
            tv_new = tv_new + jnp.sum(
                jnp.where(eq, x, 0.0), axis=0, keepdims=True
            )
        m_ref[...] = m_new
        s_ref[...] = s_new
        tv_ref[...] = tv_new

        @pl.when(j == ncb - 1)
        def _fin():
            lse_ref[...] = m_ref[...] + jnp.log(s_ref[...])
            tval_ref[...] = tv_ref[...]

    def x_spec(t):
        return pl.BlockSpec((kb, n), lambda j, t=t: (j * nstream + t, 0))

    return pl.pallas_call(
        body,
        grid=(ncb,),
        in_specs=[x_spec(t) for t in range(nstream)]
        + [pl.BlockSpec((1, n), lambda j: (0, 0))],
        out_specs=[
            pl.BlockSpec((1, n), lambda j: (0, 0)),
            pl.BlockSpec((1, n), lambda j: (0, 0)),
        ],
        out_shape=[
            jax.ShapeDtypeStruct((1, n), jnp.float32),
            jax.ShapeDtypeStruct((1, n), jnp.float32),
        ],
        scratch_shapes=[
            pltpu.VMEM((1, n), jnp.float32),
            pltpu.VMEM((1, n), jnp.float32),
            pltpu.VMEM((1, n), jnp.float32),
        ],
    )


def _make_sc_combine(n):
    info = plsc.get_sparse_core_info()
    lanes = info.num_lanes
    mesh = plsc.VectorSubcoreMesh(core_axis_name="c", subcore_axis_name="s")

    @functools.partial(
        pl.kernel,
        mesh=mesh,
        out_type=jax.ShapeDtypeStruct((lanes,), jnp.float32),
        scratch_types=[
            pltpu.VMEM((n,), jnp.float32),
            pltpu.VMEM((n,), jnp.float32),
            pltpu.VMEM((n,), jnp.float32),
            pltpu.VMEM((lanes,), jnp.float32),
        ],
    )
    def sc_combine(ratio_hbm, tval_hbm, lse_hbm, out_hbm, r_v, t_v, l_v, o_v):
        wid = lax.axis_index("s") * info.num_cores + lax.axis_index("c")

        @pl.when(wid == 0)
        def _():
            pltpu.sync_copy(ratio_hbm, r_v)
            pltpu.sync_copy(tval_hbm, t_v)
            pltpu.sync_copy(lse_hbm, l_v)
            acc = jnp.zeros((lanes,), jnp.float32)
            for k in range(n // lanes):
                sl = pl.ds(k * lanes, lanes)
                acc = acc + r_v[sl] * (t_v[sl] - l_v[sl])
            lane_ids = lax.broadcasted_iota(jnp.int32, (lanes,), 0)
            dnums = lax.GatherDimensionNumbers(
                offset_dims=(),
                collapsed_slice_dims=(0,),
                start_index_map=(0,),
            )
            step = 1
            while step < lanes:
                perm = (lane_ids ^ step).reshape(lanes, 1)
                acc = acc + lax.gather(
                    acc,
                    perm,
                    dnums,
                    (1,),
                    mode=lax.GatherScatterMode.PROMISE_IN_BOUNDS,
                )
                step *= 2
            o_v[...] = acc * (-1.0 / n)
            pltpu.sync_copy(o_v, out_hbm)

    return sc_combine


@jax.jit
def kernel(ratio, inputs, targets):
    n, c = inputs.shape
    xt = jnp.swapaxes(inputs, 0, 1)
    tgt = targets.astype(jnp.int32).reshape(1, n)
    nstream = 4
    lse, tval = _make_tc_pass(n, c, kb=1000, nstream=nstream)(
        *([xt] * nstream), tgt
    )
    out = _make_sc_combine(n)(
        ratio.reshape(n), tval.reshape(n), lse.reshape(n)
    )
    return out[0]
